# baseline (speedup 1.0000x reference)
import jax
import jax.numpy as jnp
from jax import lax
from jax.experimental import pallas as pl
from jax.experimental.pallas import tpu as pltpu
from jax.experimental.pallas import tpu_sc as plsc

BATCH = 4096
SEQ = 50
EMB = 300

NC = 2
NS = 16
NW = NC * NS
B_ROWS_PER_W = BATCH // NW
IDS_PER_W = B_ROWS_PER_W * SEQ
CB = 2
CHUNK = CB * SEQ
NCH = B_ROWS_PER_W // CB
NGRP = CHUNK // 16 + (1 if CHUNK % 16 else 0)


def _fire_chunk(table_hbm, idx_v, rows_v, sem, chunk_base):
  for m in range(NGRP):
    off = m * 16 if (m + 1) * 16 <= CHUNK else CHUNK - 16
    vec = idx_v[pl.ds(chunk_base + off, 16)]
    for j in range(m * 16 - off, min(CHUNK - off, 16)):
      p = off + j
      pltpu.async_copy(
          table_hbm.at[pl.ds(vec[j], 1)],
          rows_v.at[p // SEQ, pl.ds(p % SEQ, 1)],
          sem,
      )


DRAIN_WORDS = CHUNK * 384


def _drain_chunk(ids_hbm, dummy1d, sem):
  pltpu.make_async_copy(ids_hbm.at[pl.ds(0, DRAIN_WORDS)], dummy1d, sem).wait()


def _gather_body(ids_hbm, table_hbm, out_hbm, idx_v, rows0, rows1, dummy1d,
                 sem0, sem1):
  wid = lax.axis_index("s") * NC + lax.axis_index("c")
  base = wid * IDS_PER_W
  b_base = wid * B_ROWS_PER_W
  pltpu.sync_copy(ids_hbm.at[pl.ds(base, IDS_PER_W)], idx_v)

  _fire_chunk(table_hbm, idx_v, rows0, sem0, 0)

  @pl.loop(0, NCH, step=2)
  def _pair(c):
    _fire_chunk(table_hbm, idx_v, rows1, sem1, (c + 1) * CHUNK)
    _drain_chunk(ids_hbm, dummy1d, sem0)
    pltpu.sync_copy(rows0, out_hbm.at[pl.ds(b_base + c * CB, CB)])

    @pl.when(c + 2 < NCH)
    def _():
      _fire_chunk(table_hbm, idx_v, rows0, sem0, (c + 2) * CHUNK)

    _drain_chunk(ids_hbm, dummy1d, sem1)
    pltpu.sync_copy(rows1, out_hbm.at[pl.ds(b_base + (c + 1) * CB, CB)])


@jax.jit
def _embedding_gather(ids_flat, table):
  mesh = plsc.VectorSubcoreMesh(
      core_axis_name="c", subcore_axis_name="s", num_cores=NC, num_subcores=NS
  )
  return pl.kernel(
      _gather_body,
      out_type=jax.ShapeDtypeStruct((BATCH, SEQ, EMB), jnp.float32),
      mesh=mesh,
      scratch_types=[
          pltpu.VMEM((IDS_PER_W,), jnp.int32),
          pltpu.VMEM((CB, SEQ, EMB), jnp.float32),
          pltpu.VMEM((CB, SEQ, EMB), jnp.float32),
          pltpu.VMEM((CHUNK * 384,), jnp.int32),
          pltpu.SemaphoreType.DMA,
          pltpu.SemaphoreType.DMA,
      ],
  )(ids_flat, table)


def kernel(ids, table):
  ids_flat = ids.reshape(-1).astype(jnp.int32)
  return _embedding_gather(ids_flat, table)


# speedup vs baseline: 1.0024x; 1.0024x over previous
import jax
import jax.numpy as jnp
from jax import lax
from jax.experimental import pallas as pl
from jax.experimental.pallas import tpu as pltpu
from jax.experimental.pallas import tpu_sc as plsc

BATCH = 4096
SEQ = 50
EMB = 300

NC = 2
NS = 16
NW = NC * NS
B_ROWS_PER_W = BATCH // NW
IDS_PER_W = B_ROWS_PER_W * SEQ
CB = 1
CHUNK = CB * SEQ
NCH = B_ROWS_PER_W // CB
NGRP = CHUNK // 16 + (1 if CHUNK % 16 else 0)


def _fire_chunk(table_hbm, idx_v, rows_v, sem, chunk_base):
  for m in range(NGRP):
    off = m * 16 if (m + 1) * 16 <= CHUNK else CHUNK - 16
    vec = idx_v[pl.ds(chunk_base + off, 16)]
    for j in range(m * 16 - off, min(CHUNK - off, 16)):
      p = off + j
      pltpu.async_copy(
          table_hbm.at[pl.ds(vec[j], 1)],
          rows_v.at[p // SEQ, pl.ds(p % SEQ, 1)],
          sem,
      )


DRAIN_WORDS = CHUNK * 384


def _drain_chunk(ids_hbm, dummy1d, sem):
  pltpu.make_async_copy(ids_hbm.at[pl.ds(0, DRAIN_WORDS)], dummy1d, sem).wait()


NBUF = 4


def _gather_body(ids_hbm, table_hbm, out_hbm, idx_v, rows0, rows1, rows2,
                 rows3, dummy1d, sem0, sem1, sem2, sem3):
  rows = (rows0, rows1, rows2, rows3)
  sems = (sem0, sem1, sem2, sem3)
  wid = lax.axis_index("s") * NC + lax.axis_index("c")
  base = wid * IDS_PER_W
  b_base = wid * B_ROWS_PER_W
  pltpu.sync_copy(ids_hbm.at[pl.ds(base, IDS_PER_W)], idx_v)

  for k in range(NBUF - 1):
    _fire_chunk(table_hbm, idx_v, rows[k], sems[k], k * CHUNK)

  @pl.loop(0, NCH, step=NBUF)
  def _quad(c):
    for b in range(NBUF):
      nxt = c + b + NBUF - 1

      @pl.when(nxt < NCH)
      def _():
        _fire_chunk(table_hbm, idx_v, rows[(b + NBUF - 1) % NBUF],
                    sems[(b + NBUF - 1) % NBUF], nxt * CHUNK)

      _drain_chunk(ids_hbm, dummy1d, sems[b])
      pltpu.sync_copy(rows[b], out_hbm.at[pl.ds(b_base + (c + b) * CB, CB)])


@jax.jit
def _embedding_gather(ids_flat, table):
  mesh = plsc.VectorSubcoreMesh(
      core_axis_name="c", subcore_axis_name="s", num_cores=NC, num_subcores=NS
  )
  return pl.kernel(
      _gather_body,
      out_type=jax.ShapeDtypeStruct((BATCH, SEQ, EMB), jnp.float32),
      mesh=mesh,
      scratch_types=[
          pltpu.VMEM((IDS_PER_W,), jnp.int32),
          pltpu.VMEM((CB, SEQ, EMB), jnp.float32),
          pltpu.VMEM((CB, SEQ, EMB), jnp.float32),
          pltpu.VMEM((CB, SEQ, EMB), jnp.float32),
          pltpu.VMEM((CB, SEQ, EMB), jnp.float32),
          pltpu.VMEM((CHUNK * 384,), jnp.int32),
          pltpu.SemaphoreType.DMA,
          pltpu.SemaphoreType.DMA,
          pltpu.SemaphoreType.DMA,
          pltpu.SemaphoreType.DMA,
      ],
  )(ids_flat, table)


def kernel(ids, table):
  ids_flat = ids.reshape(-1).astype(jnp.int32)
  return _embedding_gather(ids_flat, table)
